# baseline (speedup 1.0000x reference)
import functools

import jax
import jax.numpy as jnp
from jax import lax
from jax.experimental import pallas as pl
from jax.experimental.pallas import tpu as pltpu
from jax.experimental.pallas import tpu_sc as plsc

_VOCAB = 1000
_EMB = 128
_BB = 64


def _argmax_body(x_ref, tok_ref):
    xb = x_ref[...]
    m = jnp.max(xb, axis=2, keepdims=True)
    iota = lax.broadcasted_iota(jnp.int32, xb.shape, 2)
    cand = jnp.where(xb == m, iota, _VOCAB)
    tok_ref[...] = jnp.min(cand, axis=2)


def _argmax_tokens(x):
    b, n, v = x.shape
    return pl.pallas_call(
        _argmax_body,
        grid=(b // _BB,),
        in_specs=[pl.BlockSpec((_BB, n, v), lambda i: (i, 0, 0))],
        out_specs=pl.BlockSpec((_BB, n), lambda i: (i, 0)),
        out_shape=jax.ShapeDtypeStruct((b, n), jnp.int32),
    )(x)


@functools.cache
def _make_gather(b, n):
    info = plsc.get_sparse_core_info()
    nw = info.num_cores * info.num_subcores
    b_per_w = b // nw
    mesh = plsc.VectorSubcoreMesh(core_axis_name="c", subcore_axis_name="s")

    @functools.partial(
        pl.kernel,
        mesh=mesh,
        out_type=jax.ShapeDtypeStruct((b, n, _EMB), jnp.float32),
        scratch_types=[
            pltpu.VMEM((b_per_w, n), jnp.int32),
            pltpu.VMEM((n, _EMB), jnp.float32),
            pltpu.SemaphoreType.DMA,
        ],
    )
    def gk(tok_hbm, table_hbm, out_hbm, idx_v, rows_v, sem):
        wid = lax.axis_index("s") * info.num_cores + lax.axis_index("c")
        base = wid * b_per_w
        pltpu.sync_copy(tok_hbm.at[pl.ds(base, b_per_w)], idx_v)

        def body(j, carry):
            pltpu.async_copy(table_hbm.at[idx_v.at[j]], rows_v, sem).wait()
            pltpu.sync_copy(rows_v, out_hbm.at[base + j])
            return carry

        lax.fori_loop(0, b_per_w, body, 0)

    return gk


def kernel(x, dictionary):
    b, n, v = x.shape
    tokens = _argmax_tokens(x)
    return _make_gather(b, n)(tokens, dictionary)
